# Initial kernel scaffold; baseline (speedup 1.0000x reference)
import jax
import jax.numpy as jnp
from jax.experimental import pallas as pl


def kernel(x, W_affine, b_affine, W1, W2, f1, f2):
    raise NotImplementedError("write your pallas kernel here")


# speedup vs baseline: 2.2707x; 2.2707x over previous
import functools

import jax
import jax.numpy as jnp
from jax import lax
from jax.experimental import pallas as pl
from jax.experimental.pallas import tpu as pltpu

D = 49152
N = 1024
BASE = 128
BD = 768


def _mm_body(x_ref, wt_ref, g_ref, b_ref, o_ref):
    j = pl.program_id(0)

    @pl.when(j == 0)
    def _init():
        o_ref[...] = jnp.broadcast_to(b_ref[...], o_ref.shape)

    w = wt_ref[...] + g_ref[...]
    o_ref[...] += jnp.dot(x_ref[...], w, preferred_element_type=jnp.float32)


def _matmul(x, Wt, G, b2d):
    grid = (D // BD,)
    return pl.pallas_call(
        _mm_body,
        grid=grid,
        in_specs=[
            pl.BlockSpec((N, BD), lambda j: (0, j)),
            pl.BlockSpec((BD, BASE), lambda j: (j, 0)),
            pl.BlockSpec((BD, BASE), lambda j: (j, 0)),
            pl.BlockSpec((1, BASE), lambda j: (0, 0)),
        ],
        out_specs=pl.BlockSpec((N, BASE), lambda j: (0, 0)),
        out_shape=jax.ShapeDtypeStruct((N, BASE), jnp.float32),
        compiler_params=pltpu.CompilerParams(
            dimension_semantics=("arbitrary",),
        ),
    )(x, Wt, G, b2d)


def kernel(x, W_affine, b_affine, W1, W2, f1, f2):
    Wt = W_affine.T
    G = W1[f1] + W2[f2]
    return _matmul(x, Wt, G, b_affine.reshape(1, BASE))


# speedup vs baseline: 4.1742x; 1.8383x over previous
import functools

import jax
import jax.numpy as jnp
from jax import lax
from jax.experimental import pallas as pl
from jax.experimental.pallas import tpu as pltpu
from jax.experimental.pallas import tpu_sc as plsc

D = 49152
N = 1024
BASE = 128
BD = 768

_INFO = plsc.get_sparse_core_info()
_NC, _NS, _L = _INFO.num_cores, _INFO.num_subcores, _INFO.num_lanes
_NW = _NC * _NS
_RPW = D // _NW
_CHUNK = 128
_NCHUNK = _RPW // _CHUNK


_sc_mesh = plsc.VectorSubcoreMesh(core_axis_name="c", subcore_axis_name="s")


@functools.partial(
    pl.kernel,
    mesh=_sc_mesh,
    out_type=jax.ShapeDtypeStruct((D, BASE), jnp.float32),
    scratch_types=[
        pltpu.VMEM((_RPW,), jnp.int32),
        pltpu.VMEM((_RPW,), jnp.int32),
        pltpu.VMEM((_CHUNK, BASE), jnp.float32),
        pltpu.VMEM((_CHUNK, BASE), jnp.float32),
        pltpu.SemaphoreType.DMA,
        pltpu.SemaphoreType.DMA,
    ],
)
def _sc_gather_sum(w1_hbm, w2_hbm, f1_hbm, f2_hbm, g_hbm,
                   idx1_v, idx2_v, buf1, buf2, sem1, sem2):
    wid = lax.axis_index("s") * _NC + lax.axis_index("c")
    base = wid * _RPW
    pltpu.sync_copy(f1_hbm.at[pl.ds(base, _RPW)], idx1_v)
    pltpu.sync_copy(f2_hbm.at[pl.ds(base, _RPW)], idx2_v)
    for j in range(_NCHUNK):
        off = j * _CHUNK
        c1 = pltpu.async_copy(w1_hbm.at[idx1_v.at[pl.ds(off, _CHUNK)]], buf1, sem1)
        c2 = pltpu.async_copy(w2_hbm.at[idx2_v.at[pl.ds(off, _CHUNK)]], buf2, sem2)
        c1.wait()
        c2.wait()

        def _add_row(r, _):
            for cg in range(BASE // _L):
                sl = pl.ds(cg * _L, _L)
                buf1[r, sl] += buf2[r, sl]
            return 0

        lax.fori_loop(0, _CHUNK, _add_row, 0)
        pltpu.sync_copy(buf1, g_hbm.at[pl.ds(base + off, _CHUNK)])


def _mm_body(x_ref, wt_ref, g_ref, b_ref, o_ref):
    j = pl.program_id(0)

    @pl.when(j == 0)
    def _init():
        o_ref[...] = jnp.broadcast_to(b_ref[...], o_ref.shape)

    w = wt_ref[...] + g_ref[...]
    o_ref[...] += jnp.dot(x_ref[...], w, preferred_element_type=jnp.float32)


def _matmul(x, Wt, G, b2d):
    grid = (D // BD,)
    return pl.pallas_call(
        _mm_body,
        grid=grid,
        in_specs=[
            pl.BlockSpec((N, BD), lambda j: (0, j)),
            pl.BlockSpec((BD, BASE), lambda j: (j, 0)),
            pl.BlockSpec((BD, BASE), lambda j: (j, 0)),
            pl.BlockSpec((1, BASE), lambda j: (0, 0)),
        ],
        out_specs=pl.BlockSpec((N, BASE), lambda j: (0, 0)),
        out_shape=jax.ShapeDtypeStruct((N, BASE), jnp.float32),
        compiler_params=pltpu.CompilerParams(
            dimension_semantics=("arbitrary",),
        ),
    )(x, Wt, G, b2d)


def kernel(x, W_affine, b_affine, W1, W2, f1, f2):
    Wt = W_affine.T
    G = _sc_gather_sum(W1, W2, f1, f2)
    return _matmul(x, Wt, G, b_affine.reshape(1, BASE))


# speedup vs baseline: 4.6524x; 1.1145x over previous
import functools

import jax
import jax.numpy as jnp
from jax import lax
from jax.experimental import pallas as pl
from jax.experimental.pallas import tpu as pltpu
from jax.experimental.pallas import tpu_sc as plsc

D = 49152
N = 1024
BASE = 128
BD = 768

_INFO = plsc.get_sparse_core_info()
_NC, _NS, _L = _INFO.num_cores, _INFO.num_subcores, _INFO.num_lanes
_NW = _NC * _NS
_RPW = D // _NW
_CHUNK = 128
_NCHUNK = _RPW // _CHUNK
_NBUF = 3


_sc_mesh = plsc.VectorSubcoreMesh(core_axis_name="c", subcore_axis_name="s")


@functools.partial(
    pl.kernel,
    mesh=_sc_mesh,
    out_type=(
        jax.ShapeDtypeStruct((D, BASE), jnp.float32),
        jax.ShapeDtypeStruct((D, BASE), jnp.float32),
    ),
    scratch_types=[
        pltpu.VMEM((_RPW,), jnp.int32),
        pltpu.VMEM((_RPW,), jnp.int32),
        pltpu.VMEM((_NBUF, _CHUNK, BASE), jnp.float32),
        pltpu.VMEM((_NBUF, _CHUNK, BASE), jnp.float32),
        pltpu.SemaphoreType.DMA((_NBUF,)),
        pltpu.SemaphoreType.DMA((_NBUF,)),
    ],
)
def _sc_gather(w1_hbm, w2_hbm, f1_hbm, f2_hbm, g1_hbm, g2_hbm,
               idx1_v, idx2_v, b1, b2, gsem, wsem):
    wid = lax.axis_index("s") * _NC + lax.axis_index("c")
    base = wid * _RPW
    pltpu.sync_copy(f1_hbm.at[pl.ds(base, _RPW)], idx1_v)
    pltpu.sync_copy(f2_hbm.at[pl.ds(base, _RPW)], idx2_v)

    gath = [None] * _NCHUNK
    wrt = [None] * _NCHUNK
    for j in range(_NCHUNK + 2):
        if j < _NCHUNK:
            s = j % _NBUF
            if j >= _NBUF:
                wrt[j - _NBUF][0].wait()
                wrt[j - _NBUF][1].wait()
            off = j * _CHUNK
            gath[j] = (
                pltpu.async_copy(
                    w1_hbm.at[idx1_v.at[pl.ds(off, _CHUNK)]], b1.at[s],
                    gsem.at[s]),
                pltpu.async_copy(
                    w2_hbm.at[idx2_v.at[pl.ds(off, _CHUNK)]], b2.at[s],
                    gsem.at[s]),
            )
        if j >= 2:
            k = j - 2
            s = k % _NBUF
            gath[k][0].wait()
            gath[k][1].wait()
            off = k * _CHUNK
            wrt[k] = (
                pltpu.async_copy(
                    b1.at[s], g1_hbm.at[pl.ds(base + off, _CHUNK)],
                    wsem.at[s]),
                pltpu.async_copy(
                    b2.at[s], g2_hbm.at[pl.ds(base + off, _CHUNK)],
                    wsem.at[s]),
            )
    for k in range(_NCHUNK - _NBUF, _NCHUNK):
        wrt[k][0].wait()
        wrt[k][1].wait()


def _mm_body(x_ref, wa_ref, g1_ref, g2_ref, b_ref, o_ref):
    j = pl.program_id(0)

    @pl.when(j == 0)
    def _init():
        o_ref[...] = jnp.broadcast_to(b_ref[...], o_ref.shape)

    x16 = x_ref[...].astype(jnp.bfloat16)
    wa16 = wa_ref[...].astype(jnp.bfloat16)
    g = (g1_ref[...] + g2_ref[...]).astype(jnp.bfloat16)
    acc = lax.dot_general(x16, wa16, (((1,), (1,)), ((), ())),
                          preferred_element_type=jnp.float32)
    acc += jnp.dot(x16, g, preferred_element_type=jnp.float32)
    o_ref[...] += acc


def _matmul(x, W_affine, G1, G2, b2d):
    grid = (D // BD,)
    return pl.pallas_call(
        _mm_body,
        grid=grid,
        in_specs=[
            pl.BlockSpec((N, BD), lambda j: (0, j)),
            pl.BlockSpec((BASE, BD), lambda j: (0, j)),
            pl.BlockSpec((BD, BASE), lambda j: (j, 0)),
            pl.BlockSpec((BD, BASE), lambda j: (j, 0)),
            pl.BlockSpec((1, BASE), lambda j: (0, 0)),
        ],
        out_specs=pl.BlockSpec((N, BASE), lambda j: (0, 0)),
        out_shape=jax.ShapeDtypeStruct((N, BASE), jnp.float32),
        compiler_params=pltpu.CompilerParams(
            dimension_semantics=("arbitrary",),
        ),
    )(x, W_affine, G1, G2, b2d)


def kernel(x, W_affine, b_affine, W1, W2, f1, f2):
    G1, G2 = _sc_gather(W1, W2, f1, f2)
    return _matmul(x, W_affine, G1, G2, b_affine.reshape(1, BASE))


# speedup vs baseline: 4.7621x; 1.0236x over previous
import functools

import jax
import jax.numpy as jnp
from jax import lax
from jax.experimental import pallas as pl
from jax.experimental.pallas import tpu as pltpu
from jax.experimental.pallas import tpu_sc as plsc

D = 49152
N = 1024
BASE = 128
BD = 768
NSPLIT = 4
DSPLIT = D // NSPLIT

_INFO = plsc.get_sparse_core_info()
_NC, _NS, _L = _INFO.num_cores, _INFO.num_subcores, _INFO.num_lanes
_NW = _NC * _NS
_RPW = DSPLIT // _NW
_CHUNK = 128
_NCHUNK = _RPW // _CHUNK
_NBUF = 3


_sc_mesh = plsc.VectorSubcoreMesh(core_axis_name="c", subcore_axis_name="s")


@functools.partial(
    pl.kernel,
    mesh=_sc_mesh,
    out_type=(
        jax.ShapeDtypeStruct((DSPLIT, BASE), jnp.float32),
        jax.ShapeDtypeStruct((DSPLIT, BASE), jnp.float32),
    ),
    scratch_types=[
        pltpu.VMEM((_RPW,), jnp.int32),
        pltpu.VMEM((_RPW,), jnp.int32),
        pltpu.VMEM((_NBUF, _CHUNK, BASE), jnp.float32),
        pltpu.VMEM((_NBUF, _CHUNK, BASE), jnp.float32),
        pltpu.SemaphoreType.DMA((_NBUF,)),
        pltpu.SemaphoreType.DMA((_NBUF,)),
    ],
)
def _sc_gather(w1_hbm, w2_hbm, f1_hbm, f2_hbm, g1_hbm, g2_hbm,
               idx1_v, idx2_v, b1, b2, gsem, wsem):
    wid = lax.axis_index("s") * _NC + lax.axis_index("c")
    base = wid * _RPW
    pltpu.sync_copy(f1_hbm.at[pl.ds(base, _RPW)], idx1_v)
    pltpu.sync_copy(f2_hbm.at[pl.ds(base, _RPW)], idx2_v)

    gath = [None] * _NCHUNK
    wrt = [None] * _NCHUNK
    for j in range(_NCHUNK + 2):
        if j < _NCHUNK:
            s = j % _NBUF
            if j >= _NBUF:
                wrt[j - _NBUF][0].wait()
                wrt[j - _NBUF][1].wait()
            off = j * _CHUNK
            gath[j] = (
                pltpu.async_copy(
                    w1_hbm.at[idx1_v.at[pl.ds(off, _CHUNK)]], b1.at[s],
                    gsem.at[s]),
                pltpu.async_copy(
                    w2_hbm.at[idx2_v.at[pl.ds(off, _CHUNK)]], b2.at[s],
                    gsem.at[s]),
            )
        if j >= 2:
            k = j - 2
            s = k % _NBUF
            gath[k][0].wait()
            gath[k][1].wait()
            off = k * _CHUNK
            wrt[k] = (
                pltpu.async_copy(
                    b1.at[s], g1_hbm.at[pl.ds(base + off, _CHUNK)],
                    wsem.at[s]),
                pltpu.async_copy(
                    b2.at[s], g2_hbm.at[pl.ds(base + off, _CHUNK)],
                    wsem.at[s]),
            )
    for k in range(max(_NCHUNK - _NBUF, 0), _NCHUNK):
        wrt[k][0].wait()
        wrt[k][1].wait()


def _mm_body(x_ref, wa_ref, g1_ref, g2_ref, b_ref, acc_ref, o_ref):
    j = pl.program_id(0)

    @pl.when(j == 0)
    def _init():
        o_ref[...] = acc_ref[...] + jnp.broadcast_to(b_ref[...], o_ref.shape)

    x16 = x_ref[...].astype(jnp.bfloat16)
    wa16 = wa_ref[...].astype(jnp.bfloat16)
    g = (g1_ref[...] + g2_ref[...]).astype(jnp.bfloat16)
    acc = lax.dot_general(x16, wa16, (((1,), (1,)), ((), ())),
                          preferred_element_type=jnp.float32)
    acc += jnp.dot(x16, g, preferred_element_type=jnp.float32)
    o_ref[...] += acc


def _matmul_slab(k, x, W_affine, G1, G2, bcast, acc):
    grid = (DSPLIT // BD,)
    off = k * (DSPLIT // BD)
    return pl.pallas_call(
        _mm_body,
        grid=grid,
        in_specs=[
            pl.BlockSpec((N, BD), lambda j: (0, off + j)),
            pl.BlockSpec((BASE, BD), lambda j: (0, off + j)),
            pl.BlockSpec((BD, BASE), lambda j: (j, 0)),
            pl.BlockSpec((BD, BASE), lambda j: (j, 0)),
            pl.BlockSpec((1, BASE), lambda j: (0, 0)),
            pl.BlockSpec((N, BASE), lambda j: (0, 0)),
        ],
        out_specs=pl.BlockSpec((N, BASE), lambda j: (0, 0)),
        out_shape=jax.ShapeDtypeStruct((N, BASE), jnp.float32),
        compiler_params=pltpu.CompilerParams(
            dimension_semantics=("arbitrary",),
        ),
    )(x, W_affine, G1, G2, bcast, acc)


def kernel(x, W_affine, b_affine, W1, W2, f1, f2):
    zeros = jnp.zeros((1, BASE), jnp.float32)
    gs = []
    for k in range(NSPLIT):
        sl = slice(k * DSPLIT, (k + 1) * DSPLIT)
        gs.append(_sc_gather(W1, W2, f1[sl], f2[sl]))
    acc = jnp.zeros((N, BASE), jnp.float32)
    for k in range(NSPLIT):
        bcast = b_affine.reshape(1, BASE) if k == 0 else zeros
        acc = _matmul_slab(k, x, W_affine, gs[k][0], gs[k][1], bcast, acc)
    return acc
